# Initial kernel scaffold; baseline (speedup 1.0000x reference)
import jax
import jax.numpy as jnp
from jax.experimental import pallas as pl


def kernel(x, pos_emb):
    raise NotImplementedError("write your pallas kernel here")


# speedup vs baseline: 3.4693x; 3.4693x over previous
import functools

import jax
import jax.numpy as jnp
from jax.experimental import pallas as pl
from jax.experimental.pallas import tpu as pltpu

DBLK = 256
SBLK = 2048


def _body(x_ref, pos_ref, out_ref, pe_t_ref):
    b = pl.program_id(2)

    @pl.when(b == 0)
    def _():
        pe_t_ref[...] = pos_ref[...].T

    out_ref[...] = x_ref[...] + pe_t_ref[...][None]


@jax.jit
def kernel(x, pos_emb):
    B, D, S = x.shape
    dblk = min(DBLK, D)
    sblk = min(SBLK, S)
    grid = (D // dblk, S // sblk, B)
    return pl.pallas_call(
        _body,
        grid=grid,
        in_specs=[
            pl.BlockSpec((1, dblk, sblk), lambda d, s, b: (b, d, s)),
            pl.BlockSpec((sblk, dblk), lambda d, s, b: (s, d)),
        ],
        out_specs=pl.BlockSpec((1, dblk, sblk), lambda d, s, b: (b, d, s)),
        out_shape=jax.ShapeDtypeStruct((B, D, S), x.dtype),
        scratch_shapes=[pltpu.VMEM((dblk, sblk), x.dtype)],
        compiler_params=pltpu.CompilerParams(
            dimension_semantics=("parallel", "parallel", "arbitrary"),
        ),
    )(x, pos_emb)


# speedup vs baseline: 3.8851x; 1.1199x over previous
import functools

import jax
import jax.numpy as jnp
from jax.experimental import pallas as pl
from jax.experimental.pallas import tpu as pltpu

DBLK = 512
SBLK = 2048


def _body(x_ref, pos_ref, out_ref, pe_t_ref):
    b = pl.program_id(2)

    @pl.when(b == 0)
    def _():
        pe_t_ref[...] = pos_ref[...].T

    out_ref[...] = x_ref[...] + pe_t_ref[...][None]


@jax.jit
def kernel(x, pos_emb):
    B, D, S = x.shape
    dblk = min(DBLK, D)
    sblk = min(SBLK, S)
    grid = (D // dblk, S // sblk, B)
    return pl.pallas_call(
        _body,
        grid=grid,
        in_specs=[
            pl.BlockSpec((1, dblk, sblk), lambda d, s, b: (b, d, s)),
            pl.BlockSpec((sblk, dblk), lambda d, s, b: (s, d)),
        ],
        out_specs=pl.BlockSpec((1, dblk, sblk), lambda d, s, b: (b, d, s)),
        out_shape=jax.ShapeDtypeStruct((B, D, S), x.dtype),
        scratch_shapes=[pltpu.VMEM((dblk, sblk), x.dtype)],
        compiler_params=pltpu.CompilerParams(
            dimension_semantics=("parallel", "parallel", "arbitrary"),
        ),
    )(x, pos_emb)


# speedup vs baseline: 4.3780x; 1.1269x over previous
import functools

import jax
import jax.numpy as jnp
from jax.experimental import pallas as pl
from jax.experimental.pallas import tpu as pltpu

DBLK = 256
SBLK = 2048


def _body(x_ref, pos_ref, out_ref):
    pe_t = pos_ref[...].T
    out_ref[...] = x_ref[...] + pe_t[None]


@jax.jit
def kernel(x, pos_emb):
    B, D, S = x.shape
    dblk = min(DBLK, D)
    sblk = min(SBLK, S)
    grid = (D // dblk, S // sblk)
    return pl.pallas_call(
        _body,
        grid=grid,
        in_specs=[
            pl.BlockSpec((B, dblk, sblk), lambda d, s: (0, d, s)),
            pl.BlockSpec((sblk, dblk), lambda d, s: (s, d)),
        ],
        out_specs=pl.BlockSpec((B, dblk, sblk), lambda d, s: (0, d, s)),
        out_shape=jax.ShapeDtypeStruct((B, D, S), x.dtype),
        compiler_params=pltpu.CompilerParams(
            dimension_semantics=("parallel", "parallel"),
        ),
    )(x, pos_emb)


# speedup vs baseline: 4.4298x; 1.0118x over previous
import functools

import jax
import jax.numpy as jnp
from jax.experimental import pallas as pl
from jax.experimental.pallas import tpu as pltpu

DBLK = 128
SBLK = 4096


def _body(x_ref, pos_ref, out_ref):
    pe_t = pos_ref[...].T
    out_ref[...] = x_ref[...] + pe_t[None]


@jax.jit
def kernel(x, pos_emb):
    B, D, S = x.shape
    dblk = min(DBLK, D)
    sblk = min(SBLK, S)
    grid = (D // dblk, S // sblk)
    return pl.pallas_call(
        _body,
        grid=grid,
        in_specs=[
            pl.BlockSpec((B, dblk, sblk), lambda d, s: (0, d, s)),
            pl.BlockSpec((sblk, dblk), lambda d, s: (s, d)),
        ],
        out_specs=pl.BlockSpec((B, dblk, sblk), lambda d, s: (0, d, s)),
        out_shape=jax.ShapeDtypeStruct((B, D, S), x.dtype),
        compiler_params=pltpu.CompilerParams(
            dimension_semantics=("parallel", "parallel"),
        ),
    )(x, pos_emb)
